# baseline (speedup 1.0000x reference)
import functools

import jax
import jax.numpy as jnp
from jax import lax
from jax.experimental import pallas as pl
from jax.experimental.pallas import tpu as pltpu
from jax.experimental.pallas import tpu_sc as plsc

N_NODES = 10000
N_EDGES = 320000
D_FEAT = 128
HIDDEN = 16

NC, NS = 2, 16
NW = NC * NS
LANE = 128
ER = N_EDGES // LANE
BASE_ROWS = ER // NW
EXTRA = ER - BASE_ROWS * NW
MAXR = BASE_ROWS + 1
NP = 10240
RPS = NP // NS

_mesh = plsc.VectorSubcoreMesh(core_axis_name="c", subcore_axis_name="s")
_sc_params = pltpu.CompilerParams(use_tc_tiling_on_sc=False)


def _worker_split():
    c = lax.axis_index("c")
    s = lax.axis_index("s")
    w = s * NC + c
    nrows = BASE_ROWS + jnp.where(w < EXTRA, 1, 0)
    base = BASE_ROWS * w + jnp.minimum(w, EXTRA)
    return c, s, w, nrows, base


@functools.partial(
    pl.kernel,
    out_type=jax.ShapeDtypeStruct((NC, NP), jnp.float32),
    mesh=_mesh,
    scratch_types=[
        pltpu.VMEM((MAXR, LANE), jnp.int32),
        pltpu.VMEM((LANE,), jnp.float32),
        pltpu.VMEM_SHARED((NP,), jnp.float32),
        pltpu.SemaphoreType.DMA,
    ],
    compiler_params=_sc_params,
)
def _deg_kernel(didx_hbm, zd_hbm, out_hbm, didx_v, ones_v, acc_sh, sem):
    c, s, w, nrows, base = _worker_split()

    def fill_ones(i, _):
        ones_v[pl.ds(i * 16, 16)] = jnp.full((16,), 1.0, jnp.float32)
        return 0

    lax.fori_loop(0, LANE // 16, fill_ones, 0)
    pltpu.sync_copy(zd_hbm.at[pl.ds(s * RPS, RPS)],
                    acc_sh.at[pl.ds(s * RPS, RPS)])
    plsc.subcore_barrier()

    pltpu.async_copy(didx_hbm.at[pl.ds(base, BASE_ROWS)],
                     didx_v.at[pl.ds(0, BASE_ROWS)], sem).wait()

    @pl.when(w < EXTRA)
    def _():
        pltpu.sync_copy(didx_hbm.at[base + BASE_ROWS], didx_v.at[BASE_ROWS])

    def step(j, _):
        pltpu.sync_copy(ones_v, acc_sh.at[didx_v.at[j]], add=True)
        return 0

    lax.fori_loop(0, nrows, step, 0)
    plsc.subcore_barrier()
    pltpu.sync_copy(acc_sh.at[pl.ds(s * RPS, RPS)],
                    out_hbm.at[c, pl.ds(s * RPS, RPS)])


@functools.partial(
    pl.kernel,
    out_type=jax.ShapeDtypeStruct((NC, NP, D_FEAT), jnp.float32),
    mesh=_mesh,
    scratch_types=[
        pltpu.VMEM((MAXR, LANE), jnp.int32),
        pltpu.VMEM((MAXR, LANE), jnp.int32),
        pltpu.VMEM((LANE, HIDDEN), jnp.float32),
        pltpu.VMEM((LANE, HIDDEN), jnp.float32),
        pltpu.VMEM_SHARED((NP, HIDDEN), jnp.float32),
        pltpu.SemaphoreType.DMA,
        pltpu.SemaphoreType.DMA,
        pltpu.SemaphoreType.DMA,
    ],
    compiler_params=_sc_params,
)
def _agg_kernel(g_hbm, sidx_hbm, didx_hbm, z_hbm, out_hbm,
                sidx_v, didx_v, rows0_v, rows1_v, acc_sh,
                sem_i, sem_g0, sem_g1):
    c, s, w, nrows, base = _worker_split()

    pltpu.sync_copy(z_hbm.at[pl.ds(s * RPS, RPS)],
                    acc_sh.at[pl.ds(s * RPS, RPS)])
    plsc.subcore_barrier()

    cp_s = pltpu.async_copy(sidx_hbm.at[pl.ds(base, BASE_ROWS)],
                            sidx_v.at[pl.ds(0, BASE_ROWS)], sem_i)
    cp_d = pltpu.async_copy(didx_hbm.at[pl.ds(base, BASE_ROWS)],
                            didx_v.at[pl.ds(0, BASE_ROWS)], sem_i)
    cp_s.wait()
    cp_d.wait()

    @pl.when(w < EXTRA)
    def _():
        pltpu.sync_copy(sidx_hbm.at[base + BASE_ROWS], sidx_v.at[BASE_ROWS])
        pltpu.sync_copy(didx_hbm.at[base + BASE_ROWS], didx_v.at[BASE_ROWS])

    pltpu.async_copy(g_hbm.at[sidx_v.at[0]], rows0_v, sem_g0)

    def step(i, _):
        j0 = 2 * i
        pltpu.async_copy(g_hbm.at[sidx_v.at[j0 + 1]], rows1_v, sem_g1)
        pltpu.make_async_copy(g_hbm.at[sidx_v.at[j0]], rows0_v, sem_g0).wait()
        pltpu.sync_copy(rows0_v, acc_sh.at[didx_v.at[j0]], add=True)

        @pl.when(j0 + 2 < nrows)
        def _():
            pltpu.async_copy(g_hbm.at[sidx_v.at[j0 + 2]], rows0_v, sem_g0)

        pltpu.make_async_copy(g_hbm.at[sidx_v.at[j0 + 1]], rows1_v, sem_g1).wait()
        pltpu.sync_copy(rows1_v, acc_sh.at[didx_v.at[j0 + 1]], add=True)
        return 0

    lax.fori_loop(0, BASE_ROWS // 2, step, 0)

    @pl.when(nrows > BASE_ROWS)
    def _():
        pltpu.make_async_copy(g_hbm.at[sidx_v.at[BASE_ROWS]], rows0_v, sem_g0).wait()
        pltpu.sync_copy(rows0_v, acc_sh.at[didx_v.at[BASE_ROWS]], add=True)

    plsc.subcore_barrier()
    pltpu.sync_copy(acc_sh.at[pl.ds(s * RPS, RPS)],
                    out_hbm.at[c, pl.ds(s * RPS, RPS), pl.ds(0, HIDDEN)])


def _dinv(dp_ref):
    return lax.rsqrt(dp_ref[0, :N_NODES] + dp_ref[1, :N_NODES] + 1.0)


def _tc1_body(x_ref, w1_ref, dp_ref, g1_ref):
    h = jnp.dot(x_ref[...], w1_ref[...], preferred_element_type=jnp.float32)
    g1_ref[:, pl.ds(0, HIDDEN)] = h * _dinv(dp_ref)[:, None]


def _tc2_body(agg_ref, g1_ref, dp_ref, b1_ref, w2_ref, g2_ref):
    dinv = _dinv(dp_ref)
    a = (agg_ref[0, :N_NODES, :HIDDEN] + agg_ref[1, :N_NODES, :HIDDEN]
         + g1_ref[:, :HIDDEN])
    h = jnp.maximum(a * dinv[:, None] + b1_ref[...], 0.0)
    h2 = jnp.dot(h, w2_ref[...], preferred_element_type=jnp.float32)
    g2_ref[:, pl.ds(0, HIDDEN)] = h2 * dinv[:, None]


def _tc3_body(agg_ref, g2_ref, dp_ref, b2_ref, out_ref):
    dinv = _dinv(dp_ref)
    a = (agg_ref[0, :N_NODES, :HIDDEN] + agg_ref[1, :N_NODES, :HIDDEN]
         + g2_ref[:, :HIDDEN])
    o = a * dinv[:, None] + b2_ref[...]
    m = jnp.max(o, axis=1, keepdims=True)
    e = jnp.exp(o - m)
    out_ref[...] = (o - m) - jnp.log(jnp.sum(e, axis=1, keepdims=True))


_WIDE_F32 = jax.ShapeDtypeStruct((N_NODES, D_FEAT), jnp.float32)


def kernel(x, edge_index, W1, b1, W2, b2):
    ei = edge_index.astype(jnp.int32)
    src8 = (ei[0] * 8).reshape(ER, LANE)
    dst = ei[1].reshape(ER, LANE)
    zd = jnp.zeros((NP,), jnp.float32)
    z = jnp.zeros((NP, HIDDEN), jnp.float32)
    b1r = b1.reshape(1, HIDDEN)
    b2r = b2.reshape(1, HIDDEN)

    dp = _deg_kernel(dst, zd)
    g1 = pl.pallas_call(_tc1_body, out_shape=_WIDE_F32)(x, W1, dp)
    agg1 = _agg_kernel(g1.reshape(N_NODES * 8, HIDDEN), src8, dst, z)
    g2 = pl.pallas_call(_tc2_body, out_shape=_WIDE_F32)(agg1, g1, dp, b1r, W2)
    agg2 = _agg_kernel(g2.reshape(N_NODES * 8, HIDDEN), src8, dst, z)
    out = pl.pallas_call(
        _tc3_body,
        out_shape=jax.ShapeDtypeStruct((N_NODES, HIDDEN), jnp.float32),
    )(agg2, g2, dp, b2r)
    return out
